# baseline (speedup 1.0000x reference)
import jax
import jax.numpy as jnp
from jax import lax
from jax.experimental import pallas as pl
from jax.experimental.pallas import tpu as pltpu
from jax.experimental.pallas import tpu_sc as plsc

_B = 16384
_D = 64
_NC = 2
_NS = 16
_NW = _NC * _NS
_BPW = _B // _NW
_CHUNK = 128
_NCHUNK = _BPW // _CHUNK
_GROUPS = _CHUNK // 16


def _bpr_body(user_hbm, ii_hbm, ij_hbm, eu_hbm, ei_hbm, oi_hbm, oj_hbm,
              uidx, iidx, jidx, ua, ub, via, vib, vja, vjb,
              oi_v, oj_v, sem_a, sem_b, sem_i):
    wid = lax.axis_index("s") * _NC + lax.axis_index("c")
    base = wid * _BPW

    c0 = pltpu.async_copy(user_hbm.at[wid], uidx, sem_i)
    c1 = pltpu.async_copy(ii_hbm.at[wid], iidx, sem_i)
    c2 = pltpu.async_copy(ij_hbm.at[wid], jidx, sem_i)
    c0.wait()
    c1.wait()
    c2.wait()

    rowbufs = ((ua, via, vja, sem_a), (ub, vib, vjb, sem_b))

    def issue(c, slot):
        ubuf, vibuf, vjbuf, sem = rowbufs[slot]
        return (pltpu.async_copy(eu_hbm.at[uidx.at[c]], ubuf, sem),
                pltpu.async_copy(ei_hbm.at[iidx.at[c]], vibuf, sem),
                pltpu.async_copy(ei_hbm.at[jidx.at[c]], vjbuf, sem))

    iota = lax.iota(jnp.int32, 16)

    def compute(c, slot):
        ubuf, vibuf, vjbuf, _ = rowbufs[slot]

        def group(g, carry):
            rows = g * 16 + iota
            acc_i = jnp.zeros((16,), jnp.float32)
            acc_j = jnp.zeros((16,), jnp.float32)
            for k in range(_D):
                col = jnp.full((16,), k, jnp.int32)
                u = plsc.load_gather(ubuf, [rows, col])
                vi = plsc.load_gather(vibuf, [rows, col])
                vj = plsc.load_gather(vjbuf, [rows, col])
                acc_i = acc_i + u * vi
                acc_j = acc_j + u * vj
            off = c * _CHUNK + g * 16
            oi_v[pl.ds(off, 16)] = acc_i
            oj_v[pl.ds(off, 16)] = acc_j
            return carry

        lax.fori_loop(0, _GROUPS, group, 0)

    cps = issue(0, 0)
    for c in range(_NCHUNK):
        slot = c % 2
        for cp in cps:
            cp.wait()
        if c + 1 < _NCHUNK:
            nxt = issue(c + 1, 1 - slot)
        compute(c, slot)
        if c + 1 < _NCHUNK:
            cps = nxt

    o0 = pltpu.async_copy(oi_v, oi_hbm.at[pl.ds(base, _BPW)], sem_i)
    o1 = pltpu.async_copy(oj_v, oj_hbm.at[pl.ds(base, _BPW)], sem_i)
    o0.wait()
    o1.wait()


def kernel(user, item_i, item_j, embed_user, embed_item):
    f32 = jnp.float32
    mesh = plsc.VectorSubcoreMesh(core_axis_name="c", subcore_axis_name="s")
    run = pl.kernel(
        _bpr_body,
        out_type=(jax.ShapeDtypeStruct((_B,), f32),
                  jax.ShapeDtypeStruct((_B,), f32)),
        mesh=mesh,
        compiler_params=pltpu.CompilerParams(needs_layout_passes=False, use_tc_tiling_on_sc=False),
        scratch_types=[
            pltpu.VMEM((_NCHUNK, _CHUNK), jnp.int32),
            pltpu.VMEM((_NCHUNK, _CHUNK), jnp.int32),
            pltpu.VMEM((_NCHUNK, _CHUNK), jnp.int32),
            pltpu.VMEM((_CHUNK, _D), f32),
            pltpu.VMEM((_CHUNK, _D), f32),
            pltpu.VMEM((_CHUNK, _D), f32),
            pltpu.VMEM((_CHUNK, _D), f32),
            pltpu.VMEM((_CHUNK, _D), f32),
            pltpu.VMEM((_CHUNK, _D), f32),
            pltpu.VMEM((_BPW,), f32),
            pltpu.VMEM((_BPW,), f32),
            pltpu.SemaphoreType.DMA,
            pltpu.SemaphoreType.DMA,
            pltpu.SemaphoreType.DMA,
        ],
    )
    u3 = user.reshape(_NW, _NCHUNK, _CHUNK)
    i3 = item_i.reshape(_NW, _NCHUNK, _CHUNK)
    j3 = item_j.reshape(_NW, _NCHUNK, _CHUNK)
    return run(u3, i3, j3, embed_user, embed_item)


# speedup vs baseline: 1.0055x; 1.0055x over previous
import jax
import jax.numpy as jnp
from jax import lax
from jax.experimental import pallas as pl
from jax.experimental.pallas import tpu as pltpu
from jax.experimental.pallas import tpu_sc as plsc

_B = 16384
_D = 64
_NC = 2
_NS = 16
_NW = _NC * _NS
_BPW = _B // _NW
_CHUNK = 128
_NCHUNK = _BPW // _CHUNK
_GROUPS = _CHUNK // 16


def _bpr_body(upk_hbm, ipk_hbm, jpk_hbm, uhf_hbm, ihf_hbm, jhf_hbm,
              eu_hbm, ei_hbm, oi_hbm, oj_hbm,
              upk, ipk, jpk, uhf, ihf, jhf,
              ua, ub, via, vib, vja, vjb,
              oi_v, oj_v, sem_a, sem_b, sem_i):
    wid = lax.axis_index("s") * _NC + lax.axis_index("c")
    base = wid * _BPW

    cps0 = [pltpu.async_copy(src.at[wid], dst, sem_i)
            for src, dst in ((upk_hbm, upk), (ipk_hbm, ipk), (jpk_hbm, jpk),
                             (uhf_hbm, uhf), (ihf_hbm, ihf), (jhf_hbm, jhf))]
    for cp in cps0:
        cp.wait()

    rowbufs = ((ua, via, vja, sem_a), (ub, vib, vjb, sem_b))

    def issue(c, slot):
        ubuf, vibuf, vjbuf, sem = rowbufs[slot]
        return (pltpu.async_copy(eu_hbm.at[upk.at[c]], ubuf, sem),
                pltpu.async_copy(ei_hbm.at[ipk.at[c]], vibuf, sem),
                pltpu.async_copy(ei_hbm.at[jpk.at[c]], vjbuf, sem))

    iota = lax.iota(jnp.int32, 16)

    def compute(c, slot):
        ubuf, vibuf, vjbuf, _ = rowbufs[slot]

        def group(g, carry):
            rows = g * 16 + iota
            goff = g * 16
            uoff = uhf[c, pl.ds(goff, 16)]
            ioff = ihf[c, pl.ds(goff, 16)]
            joff = jhf[c, pl.ds(goff, 16)]
            acc_i = jnp.zeros((16,), jnp.float32)
            acc_j = jnp.zeros((16,), jnp.float32)
            for k in range(_D):
                u = plsc.load_gather(ubuf, [rows, uoff + k])
                vi = plsc.load_gather(vibuf, [rows, ioff + k])
                vj = plsc.load_gather(vjbuf, [rows, joff + k])
                acc_i = acc_i + u * vi
                acc_j = acc_j + u * vj
            off = c * _CHUNK + goff
            oi_v[pl.ds(off, 16)] = acc_i
            oj_v[pl.ds(off, 16)] = acc_j
            return carry

        lax.fori_loop(0, _GROUPS, group, 0)

    cps = issue(0, 0)
    for c in range(_NCHUNK):
        slot = c % 2
        for cp in cps:
            cp.wait()
        if c + 1 < _NCHUNK:
            nxt = issue(c + 1, 1 - slot)
        compute(c, slot)
        if c + 1 < _NCHUNK:
            cps = nxt

    o0 = pltpu.async_copy(oi_v, oi_hbm.at[pl.ds(base, _BPW)], sem_i)
    o1 = pltpu.async_copy(oj_v, oj_hbm.at[pl.ds(base, _BPW)], sem_i)
    o0.wait()
    o1.wait()


def kernel(user, item_i, item_j, embed_user, embed_item):
    f32 = jnp.float32
    mesh = plsc.VectorSubcoreMesh(core_axis_name="c", subcore_axis_name="s")
    run = pl.kernel(
        _bpr_body,
        out_type=(jax.ShapeDtypeStruct((_B,), f32),
                  jax.ShapeDtypeStruct((_B,), f32)),
        mesh=mesh,
        compiler_params=pltpu.CompilerParams(needs_layout_passes=False),
        scratch_types=[
            pltpu.VMEM((_NCHUNK, _CHUNK), jnp.int32),
            pltpu.VMEM((_NCHUNK, _CHUNK), jnp.int32),
            pltpu.VMEM((_NCHUNK, _CHUNK), jnp.int32),
            pltpu.VMEM((_NCHUNK, _CHUNK), jnp.int32),
            pltpu.VMEM((_NCHUNK, _CHUNK), jnp.int32),
            pltpu.VMEM((_NCHUNK, _CHUNK), jnp.int32),
            pltpu.VMEM((_CHUNK, 2 * _D), f32),
            pltpu.VMEM((_CHUNK, 2 * _D), f32),
            pltpu.VMEM((_CHUNK, 2 * _D), f32),
            pltpu.VMEM((_CHUNK, 2 * _D), f32),
            pltpu.VMEM((_CHUNK, 2 * _D), f32),
            pltpu.VMEM((_CHUNK, 2 * _D), f32),
            pltpu.VMEM((_BPW,), f32),
            pltpu.VMEM((_BPW,), f32),
            pltpu.SemaphoreType.DMA,
            pltpu.SemaphoreType.DMA,
            pltpu.SemaphoreType.DMA,
        ],
    )
    eu2 = embed_user.reshape(embed_user.shape[0] // 2, 2 * _D)
    ei2 = embed_item.reshape(embed_item.shape[0] // 2, 2 * _D)
    shp = (_NW, _NCHUNK, _CHUNK)
    upk = (user >> 1).reshape(shp)
    ipk = (item_i >> 1).reshape(shp)
    jpk = (item_j >> 1).reshape(shp)
    uhf = ((user & 1) * _D).reshape(shp)
    ihf = ((item_i & 1) * _D).reshape(shp)
    jhf = ((item_j & 1) * _D).reshape(shp)
    return run(upk, ipk, jpk, uhf, ihf, jhf, eu2, ei2)


# speedup vs baseline: 1.0061x; 1.0006x over previous
import jax
import jax.numpy as jnp
from jax import lax
from jax.experimental import pallas as pl
from jax.experimental.pallas import tpu as pltpu
from jax.experimental.pallas import tpu_sc as plsc

_B = 16384
_D = 64
_NC = 2
_NS = 16
_NW = _NC * _NS
_BPW = _B // _NW
_CHUNK = 128
_NCHUNK = _BPW // _CHUNK
_GROUPS = _CHUNK // 16


def _bpr_body(upk_hbm, ipk_hbm, jpk_hbm, uhf_hbm, ihf_hbm, jhf_hbm,
              eu_hbm, ei_hbm, oi_hbm, oj_hbm,
              upk, ipk, jpk, uhf, ihf, jhf,
              ua, ub, via, vib, vja, vjb,
              oi_v, oj_v, sem_a, sem_b, sem_i):
    wid = lax.axis_index("s") * _NC + lax.axis_index("c")
    base = wid * _BPW

    cps0 = [pltpu.async_copy(src.at[wid], dst, sem_i)
            for src, dst in ((upk_hbm, upk), (ipk_hbm, ipk), (jpk_hbm, jpk),
                             (uhf_hbm, uhf), (ihf_hbm, ihf), (jhf_hbm, jhf))]
    for cp in cps0:
        cp.wait()

    rowbufs = ((ua, via, vja, sem_a), (ub, vib, vjb, sem_b))

    def issue(c, slot):
        ubuf, vibuf, vjbuf, sem = rowbufs[slot]
        return (pltpu.async_copy(eu_hbm.at[upk.at[c]], ubuf, sem),
                pltpu.async_copy(ei_hbm.at[ipk.at[c]], vibuf, sem),
                pltpu.async_copy(ei_hbm.at[jpk.at[c]], vjbuf, sem))

    iota = lax.iota(jnp.int32, 16)

    def compute(c, slot):
        ubuf, vibuf, vjbuf, _ = rowbufs[slot]

        def group(g, carry):
            rows = g * 16 + iota
            goff = g * 16
            uoff = uhf[c, pl.ds(goff, 16)]
            ioff = ihf[c, pl.ds(goff, 16)]
            joff = jhf[c, pl.ds(goff, 16)]
            acc_i = jnp.zeros((16,), jnp.float32)
            acc_j = jnp.zeros((16,), jnp.float32)
            for k in range(_D):
                u = plsc.load_gather(ubuf, [rows, uoff + k])
                vi = plsc.load_gather(vibuf, [rows, ioff + k])
                vj = plsc.load_gather(vjbuf, [rows, joff + k])
                acc_i = acc_i + u * vi
                acc_j = acc_j + u * vj
            off = c * _CHUNK + goff
            oi_v[pl.ds(off, 16)] = acc_i
            oj_v[pl.ds(off, 16)] = acc_j
            return carry

        lax.fori_loop(0, _GROUPS, group, 0)

    cps = issue(0, 0)
    for c in range(_NCHUNK):
        slot = c % 2
        for cp in cps:
            cp.wait()
        if c + 1 < _NCHUNK:
            nxt = issue(c + 1, 1 - slot)
        compute(c, slot)
        if c + 1 < _NCHUNK:
            cps = nxt

    o0 = pltpu.async_copy(oi_v, oi_hbm.at[pl.ds(base, _BPW)], sem_i)
    o1 = pltpu.async_copy(oj_v, oj_hbm.at[pl.ds(base, _BPW)], sem_i)
    o0.wait()
    o1.wait()


def kernel(user, item_i, item_j, embed_user, embed_item):
    f32 = jnp.float32
    mesh = plsc.VectorSubcoreMesh(core_axis_name="c", subcore_axis_name="s")
    run = pl.kernel(
        _bpr_body,
        out_type=(jax.ShapeDtypeStruct((_B,), f32),
                  jax.ShapeDtypeStruct((_B,), f32)),
        mesh=mesh,
        compiler_params=pltpu.CompilerParams(needs_layout_passes=False),
        scratch_types=[
            pltpu.VMEM((_NCHUNK, _CHUNK), jnp.int32),
            pltpu.VMEM((_NCHUNK, _CHUNK), jnp.int32),
            pltpu.VMEM((_NCHUNK, _CHUNK), jnp.int32),
            pltpu.VMEM((_NCHUNK, _CHUNK), jnp.int32),
            pltpu.VMEM((_NCHUNK, _CHUNK), jnp.int32),
            pltpu.VMEM((_NCHUNK, _CHUNK), jnp.int32),
            pltpu.VMEM((_CHUNK, 2 * _D), f32),
            pltpu.VMEM((_CHUNK, 2 * _D), f32),
            pltpu.VMEM((_CHUNK, 2 * _D), f32),
            pltpu.VMEM((_CHUNK, 2 * _D), f32),
            pltpu.VMEM((_CHUNK, 2 * _D), f32),
            pltpu.VMEM((_CHUNK, 2 * _D), f32),
            pltpu.VMEM((_BPW,), f32),
            pltpu.VMEM((_BPW,), f32),
            pltpu.SemaphoreType.DMA,
            pltpu.SemaphoreType.DMA,
            pltpu.SemaphoreType.DMA,
        ],
    )
    one = (user[0] * 0 + 1).astype(f32)
    eu2 = (embed_user * one).reshape(embed_user.shape[0] // 2, 2 * _D)
    ei2 = (embed_item * one).reshape(embed_item.shape[0] // 2, 2 * _D)
    shp = (_NW, _NCHUNK, _CHUNK)
    upk = (user >> 1).reshape(shp)
    ipk = (item_i >> 1).reshape(shp)
    jpk = (item_j >> 1).reshape(shp)
    uhf = ((user & 1) * _D).reshape(shp)
    ihf = ((item_i & 1) * _D).reshape(shp)
    jhf = ((item_j & 1) * _D).reshape(shp)
    return run(upk, ipk, jpk, uhf, ihf, jhf, eu2, ei2)
